# baseline (speedup 1.0000x reference)
import functools

import jax
import jax.numpy as jnp
from jax import lax
from jax.experimental import pallas as pl
from jax.experimental.pallas import tpu as pltpu
from jax.experimental.pallas import tpu_sc as plsc

NUM_MODELS = 8
NUM_STATES = 2048
VOCAB = 1024
BATCH = 128

NUM_CORES = 2
NUM_SUBCORES = 16
LANES = 16

NW = 16
ROWS_PER_W = BATCH // NW


_GATHER_DNUMS = lax.GatherDimensionNumbers(
    offset_dims=(), collapsed_slice_dims=(0,), start_index_map=(0,))


def _reg_gather(src, idx):
    return lax.gather(src, idx[:, None], _GATHER_DNUMS, slice_sizes=(1,),
                      mode=lax.GatherScatterMode.PROMISE_IN_BOUNDS)


def _sc_body(states_hbm, mid_hbm, scores_hbm, ns_hbm, alphas_hbm,
             scores_out, ns_out,
             st_v, md_v, idx_v, al_v, arow_v, sc_v, ns_v, sem_s, sem_n):
    c = lax.axis_index("c")
    s = lax.axis_index("s")
    wid = s * NUM_CORES + c

    @pl.when(wid < NW)
    def _():
        chunk = wid // 2
        half = wid % 2

        pltpu.sync_copy(states_hbm.at[pl.ds(chunk * LANES, LANES)], st_v)
        pltpu.sync_copy(mid_hbm.at[pl.ds(chunk * LANES, LANES)], md_v)
        pltpu.sync_copy(alphas_hbm, al_v.at[pl.ds(0, NUM_MODELS)])

        md = md_v[...]
        idx_v[...] = md * NUM_STATES + st_v[...]
        arow = _reg_gather(al_v[...], md)
        arow_v[...] = arow

        idx_slice = idx_v.at[pl.ds(half * ROWS_PER_W, ROWS_PER_W)]
        cp_s = pltpu.async_copy(scores_hbm.at[idx_slice], sc_v, sem_s)
        cp_n = pltpu.async_copy(ns_hbm.at[idx_slice], ns_v, sem_n)

        cp_s.wait()
        for r in range(ROWS_PER_W):
            lane = jnp.zeros((LANES,), jnp.int32) + (half * ROWS_PER_W + r)
            asplat = _reg_gather(arow, lane)

            def _scale(ci, _, r=r, asplat=asplat):
                sl = pl.ds(ci * LANES, LANES)
                sc_v[r, sl] = sc_v[r, sl] * asplat
                return 0

            lax.fori_loop(0, VOCAB // LANES, _scale, 0)
        cp_n.wait()

        base = wid * ROWS_PER_W
        pltpu.sync_copy(sc_v, scores_out.at[pl.ds(base, ROWS_PER_W)])
        pltpu.sync_copy(ns_v, ns_out.at[pl.ds(base, ROWS_PER_W)])


@jax.jit
def _gather_rows(states, model_ids, scores2d, ns2d, alphas):
    mesh = plsc.VectorSubcoreMesh(core_axis_name="c", subcore_axis_name="s")
    f = pl.kernel(
        _sc_body,
        out_type=(
            jax.ShapeDtypeStruct((BATCH, VOCAB), jnp.float32),
            jax.ShapeDtypeStruct((BATCH, VOCAB), jnp.int32),
        ),
        mesh=mesh,
        scratch_types=(
            pltpu.VMEM((LANES,), jnp.int32),
            pltpu.VMEM((LANES,), jnp.int32),
            pltpu.VMEM((LANES,), jnp.int32),
            pltpu.VMEM((LANES,), jnp.float32),
            pltpu.VMEM((LANES,), jnp.float32),
            pltpu.VMEM((ROWS_PER_W, VOCAB), jnp.float32),
            pltpu.VMEM((ROWS_PER_W, VOCAB), jnp.int32),
            pltpu.SemaphoreType.DMA,
            pltpu.SemaphoreType.DMA,
        ),
    )
    return f(states, model_ids, scores2d, ns2d, alphas)


def kernel(states, model_ids, scores_tables, next_states_tables, alphas):
    scores2d = scores_tables.reshape(NUM_MODELS * NUM_STATES, VOCAB)
    ns2d = next_states_tables.reshape(NUM_MODELS * NUM_STATES, VOCAB)
    return _gather_rows(states, model_ids, scores2d, ns2d, alphas)


# speedup vs baseline: 1.0463x; 1.0463x over previous
import jax
import jax.numpy as jnp
from jax import lax
from jax.experimental import pallas as pl
from jax.experimental.pallas import tpu as pltpu
from jax.experimental.pallas import tpu_sc as plsc

NUM_MODELS = 8
NUM_STATES = 2048
VOCAB = 1024
BATCH = 128

NUM_CORES = 2
NUM_SUBCORES = 16
LANES = 16

NW = 16
ROWS_PER_W = BATCH // NW

_GATHER_DNUMS = lax.GatherDimensionNumbers(
    offset_dims=(), collapsed_slice_dims=(0,), start_index_map=(0,))


def _reg_gather(src, idx):
    return lax.gather(src, idx[:, None], _GATHER_DNUMS, slice_sizes=(1,),
                      mode=lax.GatherScatterMode.PROMISE_IN_BOUNDS)


def _sc_body(states_hbm, mid_hbm, scores_hbm, ns_hbm, alphas_hbm,
             scores_out, ns_out,
             st_v, md_v, idx_v, al_v, sc_rows, ns_rows, sem_a, sem_b):
    c = lax.axis_index("c")
    s = lax.axis_index("s")
    wid = s * NUM_CORES + c
    ww = wid % NW
    chunk = ww // 2
    half = ww % 2
    base = ww * ROWS_PER_W

    cp_st = pltpu.async_copy(
        states_hbm.at[pl.ds(chunk * LANES, LANES)], st_v, sem_a)
    cp_md = pltpu.async_copy(
        mid_hbm.at[pl.ds(chunk * LANES, LANES)], md_v, sem_b)
    cp_st.wait()
    cp_md.wait()
    md = md_v[...]
    idx_v[...] = md * NUM_STATES + st_v[...]
    idx_slice = idx_v.at[pl.ds(half * ROWS_PER_W, ROWS_PER_W)]

    @pl.when(wid < NW)
    def _():
        cp_al = pltpu.async_copy(
            alphas_hbm, al_v.at[pl.ds(0, NUM_MODELS)], sem_b)
        cp = pltpu.async_copy(scores_hbm.at[idx_slice], sc_rows, sem_a)
        cp_al.wait()
        arow = _reg_gather(al_v[...], md)
        cp.wait()

        def _row(r, _):
            asplat = _reg_gather(arow, jnp.zeros((LANES,), jnp.int32)
                                 + (half * ROWS_PER_W + r))

            def _scale(ci, _):
                sl = pl.ds(ci * LANES, LANES)
                sc_rows[r, sl] = sc_rows[r, sl] * asplat
                return 0

            lax.fori_loop(0, VOCAB // LANES, _scale, 0, unroll=4)
            return 0

        lax.fori_loop(0, ROWS_PER_W, _row, 0)
        pltpu.sync_copy(sc_rows, scores_out.at[pl.ds(base, ROWS_PER_W)])

    @pl.when(wid >= NW)
    def _():
        pltpu.async_copy(ns_hbm.at[idx_slice], ns_rows, sem_a).wait()
        pltpu.sync_copy(ns_rows, ns_out.at[pl.ds(base, ROWS_PER_W)])


@jax.jit
def _gather_rows(states, model_ids, scores2d, ns2d, alphas):
    mesh = plsc.VectorSubcoreMesh(core_axis_name="c", subcore_axis_name="s")
    f = pl.kernel(
        _sc_body,
        out_type=(
            jax.ShapeDtypeStruct((BATCH, VOCAB), jnp.float32),
            jax.ShapeDtypeStruct((BATCH, VOCAB), jnp.int32),
        ),
        mesh=mesh,
        scratch_types=(
            pltpu.VMEM((LANES,), jnp.int32),
            pltpu.VMEM((LANES,), jnp.int32),
            pltpu.VMEM((LANES,), jnp.int32),
            pltpu.VMEM((LANES,), jnp.float32),
            pltpu.VMEM((ROWS_PER_W, VOCAB), jnp.float32),
            pltpu.VMEM((ROWS_PER_W, VOCAB), jnp.int32),
            pltpu.SemaphoreType.DMA,
            pltpu.SemaphoreType.DMA,
        ),
    )
    return f(states, model_ids, scores2d, ns2d, alphas)


def kernel(states, model_ids, scores_tables, next_states_tables, alphas):
    scores2d = scores_tables.reshape(NUM_MODELS * NUM_STATES, VOCAB)
    ns2d = next_states_tables.reshape(NUM_MODELS * NUM_STATES, VOCAB)
    return _gather_rows(states, model_ids, scores2d, ns2d, alphas)


# speedup vs baseline: 1.1783x; 1.1262x over previous
import functools

import jax
import jax.numpy as jnp
from jax import lax
from jax.experimental import pallas as pl
from jax.experimental.pallas import tpu as pltpu
from jax.experimental.pallas import tpu_sc as plsc

NUM_MODELS = 8
NUM_STATES = 2048
VOCAB = 1024
BATCH = 128

NUM_CORES = 2
LANES = 16

NW = 16
ROWS_PER_W = BATCH // NW


def _sc_body(states_hbm, mid_hbm, ns_hbm, ns_out,
             st_v, md_v, idx_v, ns_rows, sem_a, sem_b):
    c = lax.axis_index("c")
    s = lax.axis_index("s")
    wid = s * NUM_CORES + c

    @pl.when(wid < NW)
    def _():
        chunk = wid // 2
        half = wid % 2
        cp_st = pltpu.async_copy(
            states_hbm.at[pl.ds(chunk * LANES, LANES)], st_v, sem_a)
        cp_md = pltpu.async_copy(
            mid_hbm.at[pl.ds(chunk * LANES, LANES)], md_v, sem_b)
        cp_st.wait()
        cp_md.wait()
        idx_v[...] = md_v[...] * NUM_STATES + st_v[...]
        idx_slice = idx_v.at[pl.ds(half * ROWS_PER_W, ROWS_PER_W)]
        pltpu.async_copy(ns_hbm.at[idx_slice], ns_rows, sem_a).wait()
        pltpu.sync_copy(
            ns_rows, ns_out.at[pl.ds(wid * ROWS_PER_W, ROWS_PER_W)])


def _sc_ns(states, model_ids, ns2d):
    mesh = plsc.VectorSubcoreMesh(core_axis_name="c", subcore_axis_name="s")
    f = pl.kernel(
        _sc_body,
        out_type=jax.ShapeDtypeStruct((BATCH, VOCAB), jnp.int32),
        mesh=mesh,
        scratch_types=(
            pltpu.VMEM((LANES,), jnp.int32),
            pltpu.VMEM((LANES,), jnp.int32),
            pltpu.VMEM((LANES,), jnp.int32),
            pltpu.VMEM((ROWS_PER_W, VOCAB), jnp.int32),
            pltpu.SemaphoreType.DMA,
            pltpu.SemaphoreType.DMA,
        ),
    )
    return f(states, model_ids, ns2d)


def _tc_body(st_ref, md_ref, al_ref, md2_ref, tbl_ref, out_ref, buf, sem):
    cps = []
    for b in range(BATCH):
        idx = md_ref[b] * NUM_STATES + st_ref[b]
        cp = pltpu.make_async_copy(
            tbl_ref.at[pl.ds(idx, 1)], buf.at[pl.ds(b, 1)], sem.at[b % 8])
        cp.start()
        cps.append(cp)
    alpha = jnp.full((BATCH, 1), 0.0, dtype=jnp.float32)
    for m in range(NUM_MODELS):
        alpha = jnp.where(md2_ref[...] == m, al_ref[m], alpha)
    for cp in cps:
        cp.wait()
    out_ref[...] = buf[...] * alpha


def _tc_scores(states, model_ids, alphas, scores2d):
    md2 = model_ids.reshape(BATCH, 1)
    return pl.pallas_call(
        _tc_body,
        out_shape=jax.ShapeDtypeStruct((BATCH, VOCAB), jnp.float32),
        in_specs=[
            pl.BlockSpec(memory_space=pltpu.SMEM),
            pl.BlockSpec(memory_space=pltpu.SMEM),
            pl.BlockSpec(memory_space=pltpu.SMEM),
            pl.BlockSpec(memory_space=pltpu.VMEM),
            pl.BlockSpec(memory_space=pltpu.HBM),
        ],
        out_specs=pl.BlockSpec(memory_space=pltpu.VMEM),
        scratch_shapes=[
            pltpu.VMEM((BATCH, VOCAB), jnp.float32),
            pltpu.SemaphoreType.DMA((8,)),
        ],
    )(states, model_ids, alphas, md2, scores2d)


@jax.jit
def _run(states, model_ids, scores2d, ns2d, alphas):
    scores = _tc_scores(states, model_ids, alphas, scores2d)
    next_states = _sc_ns(states, model_ids, ns2d)
    return scores, next_states


def kernel(states, model_ids, scores_tables, next_states_tables, alphas):
    scores2d = scores_tables.reshape(NUM_MODELS * NUM_STATES, VOCAB)
    ns2d = next_states_tables.reshape(NUM_MODELS * NUM_STATES, VOCAB)
    return _run(states, model_ids, scores2d, ns2d, alphas)


# speedup vs baseline: 1.2691x; 1.0771x over previous
import functools

import jax
import jax.numpy as jnp
from jax import lax
from jax.experimental import pallas as pl
from jax.experimental.pallas import tpu as pltpu
from jax.experimental.pallas import tpu_sc as plsc

NUM_MODELS = 8
NUM_STATES = 2048
VOCAB = 1024
BATCH = 128

NUM_CORES = 2
LANES = 16

NW = 16
ROWS_PER_W = BATCH // NW


def _sc_body(states_hbm, mid_hbm, ns_hbm, ns_out,
             st_v, md_v, idx_v, ns_rows, sem_a, sem_b):
    c = lax.axis_index("c")
    s = lax.axis_index("s")
    wid = s + c * 0

    @pl.when(wid < NW)
    def _():
        chunk = wid // 2
        half = wid % 2
        cp_st = pltpu.async_copy(
            states_hbm.at[pl.ds(chunk * LANES, LANES)], st_v, sem_a)
        cp_md = pltpu.async_copy(
            mid_hbm.at[pl.ds(chunk * LANES, LANES)], md_v, sem_b)
        cp_st.wait()
        cp_md.wait()
        idx_v[...] = md_v[...] * NUM_STATES + st_v[...]
        idx_slice = idx_v.at[pl.ds(half * ROWS_PER_W, ROWS_PER_W)]
        pltpu.async_copy(ns_hbm.at[idx_slice], ns_rows, sem_a).wait()
        pltpu.sync_copy(
            ns_rows, ns_out.at[pl.ds(wid * ROWS_PER_W, ROWS_PER_W)])


def _sc_ns(states, model_ids, ns2d):
    mesh = plsc.VectorSubcoreMesh(
        core_axis_name="c", subcore_axis_name="s", num_cores=1)
    f = pl.kernel(
        _sc_body,
        out_type=jax.ShapeDtypeStruct((BATCH, VOCAB), jnp.int32),
        mesh=mesh,
        scratch_types=(
            pltpu.VMEM((LANES,), jnp.int32),
            pltpu.VMEM((LANES,), jnp.int32),
            pltpu.VMEM((LANES,), jnp.int32),
            pltpu.VMEM((ROWS_PER_W, VOCAB), jnp.int32),
            pltpu.SemaphoreType.DMA,
            pltpu.SemaphoreType.DMA,
        ),
    )
    return f(states, model_ids, ns2d)


def _tc_body(st_ref, md_ref, al_ref, md2_ref, tbl_ref, out_ref, buf, sem):
    cps = []
    for b in range(BATCH):
        idx = md_ref[b] * NUM_STATES + st_ref[b]
        cp = pltpu.make_async_copy(
            tbl_ref.at[pl.ds(idx, 1)], buf.at[pl.ds(b, 1)], sem.at[b % 8])
        cp.start()
        cps.append(cp)
    alpha = jnp.full((BATCH, 1), 0.0, dtype=jnp.float32)
    for m in range(NUM_MODELS):
        alpha = jnp.where(md2_ref[...] == m, al_ref[m], alpha)
    for cp in cps:
        cp.wait()
    out_ref[...] = buf[...] * alpha


def _tc_scores(states, model_ids, alphas, scores2d):
    md2 = model_ids.reshape(BATCH, 1)
    return pl.pallas_call(
        _tc_body,
        out_shape=jax.ShapeDtypeStruct((BATCH, VOCAB), jnp.float32),
        in_specs=[
            pl.BlockSpec(memory_space=pltpu.SMEM),
            pl.BlockSpec(memory_space=pltpu.SMEM),
            pl.BlockSpec(memory_space=pltpu.SMEM),
            pl.BlockSpec(memory_space=pltpu.VMEM),
            pl.BlockSpec(memory_space=pltpu.HBM),
        ],
        out_specs=pl.BlockSpec(memory_space=pltpu.VMEM),
        scratch_shapes=[
            pltpu.VMEM((BATCH, VOCAB), jnp.float32),
            pltpu.SemaphoreType.DMA((8,)),
        ],
    )(states, model_ids, alphas, md2, scores2d)


@jax.jit
def _run(states, model_ids, scores2d, ns2d, alphas):
    scores = _tc_scores(states, model_ids, alphas, scores2d)
    next_states = _sc_ns(states, model_ids, ns2d)
    return scores, next_states


def kernel(states, model_ids, scores_tables, next_states_tables, alphas):
    scores2d = scores_tables.reshape(NUM_MODELS * NUM_STATES, VOCAB)
    ns2d = next_states_tables.reshape(NUM_MODELS * NUM_STATES, VOCAB)
    return _run(states, model_ids, scores2d, ns2d, alphas)
